# Initial kernel scaffold; baseline (speedup 1.0000x reference)
import functools

import jax
import jax.numpy as jnp
from jax import lax
from jax.experimental import pallas as pl
from jax.experimental.pallas import tpu as pltpu
from jax.experimental.pallas import tpu_sc as plsc

NC = 2
NS = 16
L = 16

N_PRO = 50000
E_PRO = 800000
HALF = 25088
NP_PAD = 2 * HALF
DP = 64
ACC_CH = 197
ACC_ROWS = ACC_CH * 128
DUMP = HALF
E_PAD = 802816
EPT = E_PAD // NS
CHUNK = 128
NCHUNK = EPT // CHUNK

N_MOL = 4096
E_MOL = 16384
DM = 80
EPT_M = E_MOL // NS
NCHUNK_M = EPT_M // CHUNK

G = 128


def _zero_vmem_2d(ref, rows, cols):
    z = jnp.zeros((L,), jnp.float32)

    def body(i, _):
        for j in range(cols // L):
            ref[i, pl.ds(j * L, L)] = z
        return 0

    lax.fori_loop(0, rows, body, 0)


@functools.partial(
    pl.kernel,
    out_type=jax.ShapeDtypeStruct((NP_PAD, DP), jnp.float32),
    mesh=plsc.VectorSubcoreMesh(core_axis_name="c", subcore_axis_name="s"),
    scratch_types=[
        pltpu.VMEM((CHUNK,), jnp.int32),
        pltpu.VMEM((CHUNK,), jnp.int32),
        pltpu.VMEM((CHUNK,), jnp.int32),
        pltpu.VMEM((CHUNK, DP), jnp.float32),
        pltpu.VMEM_SHARED((ACC_ROWS, DP), jnp.float32),
        pltpu.SemaphoreType.DMA,
    ],
)
def _sc_conv_agg(x_hbm, src_hbm, dst_hbm, out_hbm,
                 src_v, dst_v, idx_v, msg_v, acc, sem):
    cid = lax.axis_index("c")
    sid = lax.axis_index("s")
    base_node = cid * HALF

    _zero_vmem_2d(msg_v, CHUNK, DP)
    for i in range(13):
        ch = sid + i * NS

        @pl.when(ch < ACC_CH)
        def _():
            pltpu.sync_copy(msg_v, acc.at[pl.ds(ch * 128, 128)])

    plsc.subcore_barrier()

    def body(c, _):
        off = sid * EPT + c * CHUNK
        pltpu.sync_copy(src_hbm.at[pl.ds(off, CHUNK)], src_v)
        pltpu.sync_copy(dst_hbm.at[pl.ds(off, CHUNK)], dst_v)
        for j in range(CHUNK // L):
            d = dst_v[pl.ds(j * L, L)]
            lv = d - base_node
            ok = (lv >= 0) & (lv < HALF)
            idx_v[pl.ds(j * L, L)] = jnp.where(ok, lv, DUMP)
        pltpu.async_copy(x_hbm.at[src_v], msg_v, sem).wait()
        pltpu.sync_copy(msg_v, acc.at[idx_v], add=True)
        return 0

    lax.fori_loop(0, NCHUNK, body, 0)
    plsc.subcore_barrier()

    rpt = HALF // NS
    r0 = sid * rpt
    pltpu.sync_copy(acc.at[pl.ds(r0, rpt)],
                    out_hbm.at[pl.ds(base_node + r0, rpt)])


@functools.partial(
    pl.kernel,
    out_type=tuple(jax.ShapeDtypeStruct((N_MOL, DM), jnp.float32)
                   for _ in range(4)),
    mesh=plsc.VectorSubcoreMesh(core_axis_name="c", subcore_axis_name="s"),
    scratch_types=[
        pltpu.VMEM((CHUNK,), jnp.int32),
        pltpu.VMEM((CHUNK,), jnp.int32),
        pltpu.VMEM((CHUNK, DM), jnp.float32),
        pltpu.VMEM((CHUNK, DM), jnp.float32),
        pltpu.VMEM_SHARED((N_MOL, DM), jnp.float32),
        pltpu.SemaphoreType.DMA,
    ],
)
def _sc_mol_prop(molx_hbm, src_hbm, dst_hbm, h0_hbm, h1_hbm, h2_hbm, h3_hbm,
                 src_v, dst_v, msg_v, zbuf, acc, sem):
    cid = lax.axis_index("c")
    sid = lax.axis_index("s")
    houts = [h0_hbm, h1_hbm, h2_hbm, h3_hbm]
    tables = [molx_hbm, h0_hbm, h1_hbm, h2_hbm]

    @pl.when(cid == 0)
    def _():
        _zero_vmem_2d(zbuf, CHUNK, DM)
        for k in range(4):
            pltpu.sync_copy(zbuf, acc.at[pl.ds(2 * sid * 128, 128)])
            pltpu.sync_copy(zbuf, acc.at[pl.ds((2 * sid + 1) * 128, 128)])
            plsc.subcore_barrier()

            def body(c, _, k=k):
                off = sid * EPT_M + c * CHUNK
                pltpu.sync_copy(src_hbm.at[pl.ds(off, CHUNK)], src_v)
                pltpu.sync_copy(dst_hbm.at[pl.ds(off, CHUNK)], dst_v)
                pltpu.async_copy(tables[k].at[dst_v], msg_v, sem).wait()
                pltpu.sync_copy(msg_v, acc.at[src_v], add=True)
                return 0

            lax.fori_loop(0, NCHUNK_M, body, 0)
            plsc.subcore_barrier()

            rpt = N_MOL // NS
            pltpu.sync_copy(acc.at[pl.ds(sid * rpt, rpt)],
                            houts[k].at[pl.ds(sid * rpt, rpt)])


def _tc_conv_body(agg_ref, x_ref, wl_ref, wr_ref, b_ref, out_ref):
    a = agg_ref[...]
    deg = jnp.maximum(a[:, 54:55], 1.0)
    an = a * (1.0 / deg)
    o = (jnp.dot(an, wl_ref[...], preferred_element_type=jnp.float32)
         + jnp.dot(x_ref[...], wr_ref[...], preferred_element_type=jnp.float32)
         + b_ref[...])
    out_ref[...] = jnp.maximum(o, 0.0)


def _tc_conv(agg, x, wlt, wrt, bias):
    n = agg.shape[0]
    bn = 2048
    grid = n // bn
    return pl.pallas_call(
        _tc_conv_body,
        grid=(grid,),
        in_specs=[
            pl.BlockSpec((bn, DP), lambda i: (i, 0)),
            pl.BlockSpec((bn, DP), lambda i: (i, 0)),
            pl.BlockSpec((DP, DP), lambda i: (0, 0)),
            pl.BlockSpec((DP, DP), lambda i: (0, 0)),
            pl.BlockSpec((1, DP), lambda i: (0, 0)),
        ],
        out_specs=pl.BlockSpec((bn, DP), lambda i: (i, 0)),
        out_shape=jax.ShapeDtypeStruct((n, DP), jnp.float32),
    )(agg, x, wlt, wrt, bias)


def _tc_conv2_pool_body(agg_ref, x_ref, b3_ref, wl_ref, wr_ref, bias_ref,
                        out_ref):
    i = pl.program_id(0)
    a = agg_ref[...]
    deg = jnp.maximum(a[:, 54:55], 1.0)
    an = a * (1.0 / deg)
    xt2 = jnp.maximum(
        jnp.dot(an, wl_ref[...], preferred_element_type=jnp.float32)
        + jnp.dot(x_ref[...], wr_ref[...], preferred_element_type=jnp.float32)
        + bias_ref[...], 0.0)
    b = b3_ref[0]
    oh = (jax.lax.broadcasted_iota(jnp.int32, (G, b.shape[1]), 0)
          == b).astype(jnp.float32)
    contrib = jnp.dot(oh, xt2, preferred_element_type=jnp.float32)

    @pl.when(i == 0)
    def _():
        out_ref[...] = jnp.zeros_like(out_ref)

    out_ref[...] += contrib


def _tc_conv2_pool(agg, x, batch3, wlt, wrt, bias):
    n = agg.shape[0]
    bn = 2048
    grid = n // bn
    return pl.pallas_call(
        _tc_conv2_pool_body,
        grid=(grid,),
        in_specs=[
            pl.BlockSpec((bn, DP), lambda i: (i, 0)),
            pl.BlockSpec((bn, DP), lambda i: (i, 0)),
            pl.BlockSpec((1, 1, bn), lambda i: (i, 0, 0)),
            pl.BlockSpec((DP, G), lambda i: (0, 0)),
            pl.BlockSpec((DP, G), lambda i: (0, 0)),
            pl.BlockSpec((1, G), lambda i: (0, 0)),
        ],
        out_specs=pl.BlockSpec((G, G), lambda i: (0, 0)),
        out_shape=jax.ShapeDtypeStruct((G, G), jnp.float32),
    )(agg, x, batch3, wlt, wrt, bias)


def _tc_final_body(h0, h1, h2, h3, molx, molb, pro_pool,
                   mg1, mb1, mg2, mb2, pg1, pb1, pg2, pb2,
                   f1a, f1b, b1, f2, b2, ow, ob, out_ref):
    alpha = 0.05
    q = (1.0 - alpha) / 4.0
    emb = alpha * molx[...] + q * (h0[...] + h1[...] + h2[...] + h3[...])
    colmask = jax.lax.broadcasted_iota(jnp.int32, (N_MOL, DM), 1) == 78
    emb = jnp.where(colmask, 1.0, emb)
    ohT = (molb[...] == jax.lax.broadcasted_iota(jnp.int32, (N_MOL, G), 1)
           ).astype(jnp.float32)
    pm = jax.lax.dot_general(ohT, emb, (((0,), (0,)), ((), ())),
                             preferred_element_type=jnp.float32)
    pm = pm * (1.0 / jnp.maximum(pm[:, 78:79], 1.0))
    xm = jnp.maximum(jnp.dot(pm, mg1[...],
                             preferred_element_type=jnp.float32) + mb1[...],
                     0.0)
    xm = jnp.dot(xm, mg2[...], preferred_element_type=jnp.float32) + mb2[...]

    pp = pro_pool[...]
    pp = pp * (1.0 / jnp.maximum(pp[:, 108:109], 1.0))
    xp = jnp.maximum(jnp.dot(pp, pg1[...],
                             preferred_element_type=jnp.float32) + pb1[...],
                     0.0)
    xp = jnp.dot(xp, pg2[...], preferred_element_type=jnp.float32) + pb2[...]

    h = jnp.maximum(jnp.dot(xm, f1a[...], preferred_element_type=jnp.float32)
                    + jnp.dot(xp, f1b[...],
                              preferred_element_type=jnp.float32)
                    + b1[...], 0.0)
    h = jnp.maximum(jnp.dot(h, f2[...], preferred_element_type=jnp.float32)
                    + b2[...], 0.0)
    out_ref[...] = (jnp.dot(h, ow[...], preferred_element_type=jnp.float32)
                    + ob[...])


def _tc_final(hs, molx, molb2, pro_pool, mg1, mb1, mg2, mb2,
              pg1, pb1, pg2, pb2, f1a, f1b, b1, f2, b2, ow, ob):
    return pl.pallas_call(
        _tc_final_body,
        out_shape=jax.ShapeDtypeStruct((G, 1), jnp.float32),
    )(hs[0], hs[1], hs[2], hs[3], molx, molb2, pro_pool,
      mg1, mb1, mg2, mb2, pg1, pb1, pg2, pb2, f1a, f1b, b1, f2, b2, ow, ob)


def kernel(mol_x, mol_edge_index, mol_batch, target_x, target_edge_index,
           target_batch, mol_fc_g1_W, mol_fc_g1_b, mol_fc_g2_W, mol_fc_g2_b,
           pro_conv1_Wl, pro_conv1_Wr, pro_conv1_b, pro_conv2_Wl,
           pro_conv2_Wr, pro_conv2_b, pro_fc_g1_W, pro_fc_g1_b, pro_fc_g2_W,
           pro_fc_g2_b, fc1_W, fc1_b, fc2_W, fc2_b, out_W, out_b):
    f32 = jnp.float32

    n_pro = target_x.shape[0]
    x_pad = jnp.concatenate(
        [target_x, jnp.ones((n_pro, 1), f32), jnp.zeros((n_pro, DP - 55), f32)],
        axis=1)
    x_pad = jnp.pad(x_pad, ((0, NP_PAD - n_pro), (0, 0)))
    src_pad = jnp.pad(target_edge_index[0], (0, E_PAD - E_PRO))
    dst_pad = jnp.pad(target_edge_index[1], (0, E_PAD - E_PRO),
                      constant_values=-1)
    tb3 = jnp.pad(target_batch, (0, NP_PAD - n_pro),
                  constant_values=-1).reshape(NP_PAD // 2048, 1, 2048)

    molx_pad = jnp.pad(mol_x, ((0, 0), (0, DM - mol_x.shape[1])))
    molb2 = mol_batch.reshape(N_MOL, 1)

    def padw(w, o, i):
        return jnp.pad(w, ((0, o - w.shape[0]), (0, i - w.shape[1]))).T

    wl1 = padw(pro_conv1_Wl, DP, DP)
    wr1 = padw(pro_conv1_Wr, DP, DP)
    b1c = jnp.zeros((DP,), f32).at[:54].set(pro_conv1_b).at[54].set(1.0)
    wl2 = padw(pro_conv2_Wl, G, DP)
    wr2 = padw(pro_conv2_Wr, G, DP)
    b2c = jnp.zeros((G,), f32).at[:108].set(pro_conv2_b).at[108].set(1.0)

    mg1 = jnp.pad(mol_fc_g1_W, ((0, 0), (0, DM - 78))).T
    mg2 = mol_fc_g2_W.T
    pg1 = jnp.pad(pro_fc_g1_W, ((0, 0), (0, G - 108))).T
    pg2 = pro_fc_g2_W.T
    f1a = fc1_W[:, :112].T
    f1b = fc1_W[:, 112:].T
    f2 = fc2_W.T
    ow = out_W.T

    row = lambda v: v.reshape(1, -1)

    hs = _sc_mol_prop(molx_pad, mol_edge_index[0], mol_edge_index[1])

    agg1 = _sc_conv_agg(x_pad, src_pad, dst_pad)
    xt1 = _tc_conv(agg1, x_pad, wl1, wr1, row(b1c))
    agg2 = _sc_conv_agg(xt1, src_pad, dst_pad)
    pro_pool = _tc_conv2_pool(agg2, xt1, tb3, wl2, wr2, row(b2c))

    return _tc_final(hs, molx_pad, molb2, pro_pool,
                     mg1, row(mol_fc_g1_b), mg2, row(mol_fc_g2_b),
                     pg1, row(pro_fc_g1_b), pg2, row(pro_fc_g2_b),
                     f1a, f1b, row(fc1_b), f2, row(fc2_b),
                     ow, out_b.reshape(1, 1))


# speedup vs baseline: 4.2941x; 4.2941x over previous
import functools

import jax
import jax.numpy as jnp
from jax import lax
from jax.experimental import pallas as pl
from jax.experimental.pallas import tpu as pltpu
from jax.experimental.pallas import tpu_sc as plsc

NC = 2
NS = 16
L = 16

N_PRO = 50000
E_PRO = 800000
HALF = 25088
NP_PAD = 2 * HALF
DP = 64
ACC_CH = 197
ACC_ROWS = ACC_CH * 128
DUMP = HALF
E_PAD = 802816
EPT = E_PAD // NS
CHUNK = 128
NCHUNK = EPT // CHUNK

N_MOL = 4096
E_MOL = 16384
DM = 80
EPT_M = E_MOL // NS
NCHUNK_M = EPT_M // CHUNK

G = 128


def _zero_vmem_2d(ref, rows, cols):
    z = jnp.zeros((L,), jnp.float32)

    def body(i, _):
        for j in range(cols // L):
            ref[i, pl.ds(j * L, L)] = z
        return 0

    lax.fori_loop(0, rows, body, 0)


@functools.cache
def _build_sc_conv_agg():
  return functools.partial(
      pl.kernel,
      out_type=jax.ShapeDtypeStruct((NP_PAD, DP), jnp.float32),
      mesh=plsc.VectorSubcoreMesh(core_axis_name="c", subcore_axis_name="s",
                                  num_cores=NC, num_subcores=NS),
      scratch_types=[
          pltpu.VMEM((CHUNK,), jnp.int32),
          pltpu.VMEM((CHUNK,), jnp.int32),
          pltpu.VMEM((CHUNK,), jnp.int32),
          pltpu.VMEM((CHUNK, DP), jnp.float32),
          pltpu.VMEM_SHARED((ACC_ROWS, DP), jnp.float32),
          pltpu.SemaphoreType.DMA,
      ],
      compiler_params=pltpu.CompilerParams(use_tc_tiling_on_sc=False),
  )(_sc_conv_agg_body)


def _sc_conv_agg(x_pad, src, dst):
  return _build_sc_conv_agg()(x_pad, src, dst)


def _sc_conv_agg_body(x_hbm, src_hbm, dst_hbm, out_hbm,
                      src_v, dst_v, idx_v, msg_v, acc, sem):
    cid = lax.axis_index("c")
    sid = lax.axis_index("s")
    base_node = cid * HALF

    _zero_vmem_2d(msg_v, CHUNK, DP)
    for i in range(13):
        ch = sid + i * NS

        @pl.when(ch < ACC_CH)
        def _():
            pltpu.sync_copy(msg_v, acc.at[pl.ds(ch * 128, 128)])

    plsc.subcore_barrier()

    def body(c, _):
        off = sid * EPT + c * CHUNK
        pltpu.sync_copy(src_hbm.at[pl.ds(off, CHUNK)], src_v)
        pltpu.sync_copy(dst_hbm.at[pl.ds(off, CHUNK)], dst_v)
        for j in range(CHUNK // L):
            d = dst_v[pl.ds(j * L, L)]
            lv = d - base_node
            ok = (lv >= 0) & (lv < HALF)
            idx_v[pl.ds(j * L, L)] = jnp.where(ok, lv, DUMP)
        pltpu.async_copy(x_hbm.at[src_v], msg_v, sem).wait()
        pltpu.sync_copy(msg_v, acc.at[idx_v], add=True)
        return 0

    lax.fori_loop(0, NCHUNK, body, 0)
    plsc.subcore_barrier()

    rpt = HALF // NS
    r0 = sid * rpt
    pltpu.sync_copy(acc.at[pl.ds(r0, rpt)],
                    out_hbm.at[pl.ds(base_node + r0, rpt)])


@functools.cache
def _build_sc_mol_prop():
  return functools.partial(
      pl.kernel,
      out_type=tuple(jax.ShapeDtypeStruct((N_MOL, DM), jnp.float32)
                     for _ in range(4)),
      mesh=plsc.VectorSubcoreMesh(core_axis_name="c", subcore_axis_name="s",
                                  num_cores=NC, num_subcores=NS),
      scratch_types=[
          pltpu.VMEM((CHUNK,), jnp.int32),
          pltpu.VMEM((CHUNK,), jnp.int32),
          pltpu.VMEM((CHUNK, DM), jnp.float32),
          pltpu.VMEM((CHUNK, DM), jnp.float32),
          pltpu.VMEM_SHARED((N_MOL, DM), jnp.float32),
          pltpu.SemaphoreType.DMA,
      ],
      compiler_params=pltpu.CompilerParams(use_tc_tiling_on_sc=False),
  )(_sc_mol_prop_body)


def _sc_mol_prop(molx_pad, src, dst):
  return _build_sc_mol_prop()(molx_pad, src, dst)


def _sc_mol_prop_body(molx_hbm, src_hbm, dst_hbm, h0_hbm, h1_hbm, h2_hbm,
                      h3_hbm, src_v, dst_v, msg_v, zbuf, acc, sem):
    cid = lax.axis_index("c")
    sid = lax.axis_index("s")
    houts = [h0_hbm, h1_hbm, h2_hbm, h3_hbm]
    tables = [molx_hbm, h0_hbm, h1_hbm, h2_hbm]

    @pl.when(cid == 0)
    def _():
        _zero_vmem_2d(zbuf, CHUNK, DM)
        for k in range(4):
            pltpu.sync_copy(zbuf, acc.at[pl.ds(2 * sid * 128, 128)])
            pltpu.sync_copy(zbuf, acc.at[pl.ds((2 * sid + 1) * 128, 128)])
            plsc.subcore_barrier()

            def body(c, _, k=k):
                off = sid * EPT_M + c * CHUNK
                pltpu.sync_copy(src_hbm.at[pl.ds(off, CHUNK)], src_v)
                pltpu.sync_copy(dst_hbm.at[pl.ds(off, CHUNK)], dst_v)
                pltpu.async_copy(tables[k].at[dst_v], msg_v, sem).wait()
                pltpu.sync_copy(msg_v, acc.at[src_v], add=True)
                return 0

            lax.fori_loop(0, NCHUNK_M, body, 0)
            plsc.subcore_barrier()

            rpt = N_MOL // NS
            pltpu.sync_copy(acc.at[pl.ds(sid * rpt, rpt)],
                            houts[k].at[pl.ds(sid * rpt, rpt)])


def _tc_conv_body(agg_ref, x_ref, wl_ref, wr_ref, b_ref, out_ref):
    a = agg_ref[...]
    deg = jnp.maximum(a[:, 54:55], 1.0)
    an = a * (1.0 / deg)
    o = (jnp.dot(an, wl_ref[...], preferred_element_type=jnp.float32)
         + jnp.dot(x_ref[...], wr_ref[...], preferred_element_type=jnp.float32)
         + b_ref[...])
    out_ref[...] = jnp.maximum(o, 0.0)


def _tc_conv(agg, x, wlt, wrt, bias):
    n = agg.shape[0]
    bn = 1024
    grid = n // bn
    return pl.pallas_call(
        _tc_conv_body,
        grid=(grid,),
        in_specs=[
            pl.BlockSpec((bn, DP), lambda i: (i, 0)),
            pl.BlockSpec((bn, DP), lambda i: (i, 0)),
            pl.BlockSpec((DP, DP), lambda i: (0, 0)),
            pl.BlockSpec((DP, DP), lambda i: (0, 0)),
            pl.BlockSpec((1, DP), lambda i: (0, 0)),
        ],
        out_specs=pl.BlockSpec((bn, DP), lambda i: (i, 0)),
        out_shape=jax.ShapeDtypeStruct((n, DP), jnp.float32),
    )(agg, x, wlt, wrt, bias)


def _tc_conv2_pool_body(agg_ref, x_ref, b3_ref, wl_ref, wr_ref, bias_ref,
                        out_ref):
    i = pl.program_id(0)
    a = agg_ref[...]
    deg = jnp.maximum(a[:, 54:55], 1.0)
    an = a * (1.0 / deg)
    xt2 = jnp.maximum(
        jnp.dot(an, wl_ref[...], preferred_element_type=jnp.float32)
        + jnp.dot(x_ref[...], wr_ref[...], preferred_element_type=jnp.float32)
        + bias_ref[...], 0.0)
    b = b3_ref[0]
    oh = (jax.lax.broadcasted_iota(jnp.int32, (G, b.shape[1]), 0)
          == b).astype(jnp.float32)
    contrib = jnp.dot(oh, xt2, preferred_element_type=jnp.float32)

    @pl.when(i == 0)
    def _():
        out_ref[...] = jnp.zeros_like(out_ref)

    out_ref[...] += contrib


def _tc_conv2_pool(agg, x, batch3, wlt, wrt, bias):
    n = agg.shape[0]
    bn = 1024
    grid = n // bn
    return pl.pallas_call(
        _tc_conv2_pool_body,
        grid=(grid,),
        in_specs=[
            pl.BlockSpec((bn, DP), lambda i: (i, 0)),
            pl.BlockSpec((bn, DP), lambda i: (i, 0)),
            pl.BlockSpec((1, 1, bn), lambda i: (i, 0, 0)),
            pl.BlockSpec((DP, G), lambda i: (0, 0)),
            pl.BlockSpec((DP, G), lambda i: (0, 0)),
            pl.BlockSpec((1, G), lambda i: (0, 0)),
        ],
        out_specs=pl.BlockSpec((G, G), lambda i: (0, 0)),
        out_shape=jax.ShapeDtypeStruct((G, G), jnp.float32),
    )(agg, x, batch3, wlt, wrt, bias)


def _tc_final_body(h0, h1, h2, h3, molx, molb, pro_pool,
                   mg1, mb1, mg2, mb2, pg1, pb1, pg2, pb2,
                   f1a, f1b, b1, f2, b2, ow, ob, out_ref):
    alpha = 0.05
    q = (1.0 - alpha) / 4.0
    emb = alpha * molx[...] + q * (h0[...] + h1[...] + h2[...] + h3[...])
    colmask = jax.lax.broadcasted_iota(jnp.int32, (N_MOL, DM), 1) == 78
    emb = jnp.where(colmask, 1.0, emb)
    ohT = (molb[...] == jax.lax.broadcasted_iota(jnp.int32, (N_MOL, G), 1)
           ).astype(jnp.float32)
    pm = jax.lax.dot_general(ohT, emb, (((0,), (0,)), ((), ())),
                             preferred_element_type=jnp.float32)
    pm = pm * (1.0 / jnp.maximum(pm[:, 78:79], 1.0))
    xm = jnp.maximum(jnp.dot(pm, mg1[...],
                             preferred_element_type=jnp.float32) + mb1[...],
                     0.0)
    xm = jnp.dot(xm, mg2[...], preferred_element_type=jnp.float32) + mb2[...]

    pp = pro_pool[...]
    pp = pp * (1.0 / jnp.maximum(pp[:, 108:109], 1.0))
    xp = jnp.maximum(jnp.dot(pp, pg1[...],
                             preferred_element_type=jnp.float32) + pb1[...],
                     0.0)
    xp = jnp.dot(xp, pg2[...], preferred_element_type=jnp.float32) + pb2[...]

    h = jnp.maximum(jnp.dot(xm, f1a[...], preferred_element_type=jnp.float32)
                    + jnp.dot(xp, f1b[...],
                              preferred_element_type=jnp.float32)
                    + b1[...], 0.0)
    h = jnp.maximum(jnp.dot(h, f2[...], preferred_element_type=jnp.float32)
                    + b2[...], 0.0)
    out_ref[...] = (jnp.dot(h, ow[...], preferred_element_type=jnp.float32)
                    + ob[...])


def _tc_final(hs, molx, molb2, pro_pool, mg1, mb1, mg2, mb2,
              pg1, pb1, pg2, pb2, f1a, f1b, b1, f2, b2, ow, ob):
    return pl.pallas_call(
        _tc_final_body,
        out_shape=jax.ShapeDtypeStruct((G, 1), jnp.float32),
    )(hs[0], hs[1], hs[2], hs[3], molx, molb2, pro_pool,
      mg1, mb1, mg2, mb2, pg1, pb1, pg2, pb2, f1a, f1b, b1, f2, b2, ow, ob)


def kernel(mol_x, mol_edge_index, mol_batch, target_x, target_edge_index,
           target_batch, mol_fc_g1_W, mol_fc_g1_b, mol_fc_g2_W, mol_fc_g2_b,
           pro_conv1_Wl, pro_conv1_Wr, pro_conv1_b, pro_conv2_Wl,
           pro_conv2_Wr, pro_conv2_b, pro_fc_g1_W, pro_fc_g1_b, pro_fc_g2_W,
           pro_fc_g2_b, fc1_W, fc1_b, fc2_W, fc2_b, out_W, out_b):
    f32 = jnp.float32

    n_pro = target_x.shape[0]
    x_pad = jnp.concatenate(
        [target_x, jnp.ones((n_pro, 1), f32), jnp.zeros((n_pro, DP - 55), f32)],
        axis=1)
    x_pad = jnp.pad(x_pad, ((0, NP_PAD - n_pro), (0, 0)))
    src_pad = jnp.pad(target_edge_index[0], (0, E_PAD - E_PRO))
    dst_pad = jnp.pad(target_edge_index[1], (0, E_PAD - E_PRO),
                      constant_values=-1)
    tb3 = jnp.pad(target_batch, (0, NP_PAD - n_pro),
                  constant_values=-1).reshape(NP_PAD // 1024, 1, 1024)

    molx_pad = jnp.pad(mol_x, ((0, 0), (0, DM - mol_x.shape[1])))
    molb2 = mol_batch.reshape(N_MOL, 1)

    def padw(w, o, i):
        return jnp.pad(w, ((0, o - w.shape[0]), (0, i - w.shape[1]))).T

    wl1 = padw(pro_conv1_Wl, DP, DP)
    wr1 = padw(pro_conv1_Wr, DP, DP)
    b1c = jnp.zeros((DP,), f32).at[:54].set(pro_conv1_b).at[54].set(1.0)
    wl2 = padw(pro_conv2_Wl, G, DP)
    wr2 = padw(pro_conv2_Wr, G, DP)
    b2c = jnp.zeros((G,), f32).at[:108].set(pro_conv2_b).at[108].set(1.0)

    mg1 = jnp.pad(mol_fc_g1_W, ((0, 0), (0, DM - 78))).T
    mg2 = mol_fc_g2_W.T
    pg1 = jnp.pad(pro_fc_g1_W, ((0, 0), (0, G - 108))).T
    pg2 = pro_fc_g2_W.T
    f1a = fc1_W[:, :112].T
    f1b = fc1_W[:, 112:].T
    f2 = fc2_W.T
    ow = out_W.T

    row = lambda v: v.reshape(1, -1)

    hs = _sc_mol_prop(molx_pad, mol_edge_index[0], mol_edge_index[1])

    agg1 = _sc_conv_agg(x_pad, src_pad, dst_pad)
    xt1 = _tc_conv(agg1, x_pad, wl1, wr1, row(b1c))
    agg2 = _sc_conv_agg(xt1, src_pad, dst_pad)
    pro_pool = _tc_conv2_pool(agg2, xt1, tb3, wl2, wr2, row(b2c))

    return _tc_final(hs, molx_pad, molb2, pro_pool,
                     mg1, row(mol_fc_g1_b), mg2, row(mol_fc_g2_b),
                     pg1, row(pro_fc_g1_b), pg2, row(pro_fc_g2_b),
                     f1a, f1b, row(fc1_b), f2, row(fc2_b),
                     ow, out_b.reshape(1, 1))


# speedup vs baseline: 7.4029x; 1.7240x over previous
import functools

import jax
import jax.numpy as jnp
from jax import lax
from jax.experimental import pallas as pl
from jax.experimental.pallas import tpu as pltpu
from jax.experimental.pallas import tpu_sc as plsc

NC = 2
NS = 16
L = 16

N_PRO = 50000
E_PRO = 800000
HALF = 25088
NP_PAD = 2 * HALF
DP = 64
ACC_CH = 197
ACC_ROWS = ACC_CH * 128
DUMP = HALF
E_PAD = 802816
EPT = E_PAD // NS
CHUNK = 128
NCHUNK = EPT // CHUNK
SUP = 8
SUPA = 4
NSUPA = NCHUNK // SUPA
DPH = DP // 2
ACC2_CH = 393
ACC2_ROWS = ACC2_CH * 128
DUMP2 = NP_PAD

N_MOL = 4096
E_MOL = 16384
DM = 80
EPT_M = E_MOL // NS
NCHUNK_M = EPT_M // CHUNK

G = 128


def _zero_vmem_2d(ref, rows, cols):
    z = jnp.zeros((L,), jnp.float32)

    def body(i, _):
        for j in range(cols // L):
            ref[i, pl.ds(j * L, L)] = z
        return 0

    lax.fori_loop(0, rows, body, 0)


def _zero_vmem_3d0(ref, rows, cols):
    z = jnp.zeros((L,), jnp.float32)

    def body(i, _):
        for j in range(cols // L):
            ref[0, i, pl.ds(j * L, L)] = z
        return 0

    lax.fori_loop(0, rows, body, 0)


@functools.cache
def _build_sc_conv_agg():
  return functools.partial(
      pl.kernel,
      out_type=jax.ShapeDtypeStruct((2 * NP_PAD, DPH), jnp.float32),
      mesh=plsc.VectorSubcoreMesh(core_axis_name="c", subcore_axis_name="s",
                                  num_cores=NC, num_subcores=NS),
      scratch_types=[
          pltpu.VMEM((SUPA, CHUNK), jnp.int32),
          pltpu.VMEM((SUPA, CHUNK), jnp.int32),
          pltpu.VMEM((SUPA, CHUNK), jnp.int32),
          pltpu.VMEM((SUPA, CHUNK), jnp.int32),
          pltpu.VMEM((SUPA, CHUNK, DPH), jnp.float32),
          pltpu.VMEM_SHARED((ACC2_ROWS, DPH), jnp.float32),
          pltpu.SemaphoreType.DMA,
          pltpu.SemaphoreType.DMA,
      ],
      compiler_params=pltpu.CompilerParams(use_tc_tiling_on_sc=False),
  )(_sc_conv_agg_body)


def _sc_conv_agg(xcat, src, dst):
  return _build_sc_conv_agg()(xcat, src, dst)


def _sc_conv_agg_body(x_hbm, src_hbm, dst_hbm, out_hbm,
                      src_v, dst_v, gidx_v, sidx_v, msg_v, acc, sem, sem2):
    cid = lax.axis_index("c")
    sid = lax.axis_index("s")
    goff = cid * NP_PAD

    _zero_vmem_3d0(msg_v, CHUNK, DPH)
    for i in range(25):
        ch = sid + i * NS

        @pl.when(ch < ACC2_CH)
        def _():
            pltpu.sync_copy(msg_v.at[0], acc.at[pl.ds(ch * 128, 128)])

    plsc.subcore_barrier()

    def body(s, _):
        row = sid * NCHUNK + s * SUPA
        pltpu.sync_copy(src_hbm.at[pl.ds(row, SUPA)], src_v)
        pltpu.sync_copy(dst_hbm.at[pl.ds(row, SUPA)], dst_v)
        for b in range(SUPA):
            for j in range(CHUNK // L):
                gidx_v[b, pl.ds(j * L, L)] = (
                    src_v[b, pl.ds(j * L, L)] + goff)
                d = dst_v[b, pl.ds(j * L, L)]
                sidx_v[b, pl.ds(j * L, L)] = jnp.where(d >= 0, d, DUMP2)
        gd = [pltpu.async_copy(x_hbm.at[gidx_v.at[b]], msg_v.at[b], sem)
              for b in range(SUPA)]
        for g in gd:
            g.wait()
        sd = [pltpu.async_copy(msg_v.at[b], acc.at[sidx_v.at[b]], sem2,
                               add=True)
              for b in range(SUPA)]
        for d_ in sd:
            d_.wait()
        return 0

    lax.fori_loop(0, NSUPA, body, 0)
    plsc.subcore_barrier()

    rpt = NP_PAD // NS
    r0 = sid * rpt
    pltpu.sync_copy(acc.at[pl.ds(r0, rpt)],
                    out_hbm.at[pl.ds(goff + r0, rpt)])


@functools.cache
def _build_sc_mol_prop():
  return functools.partial(
      pl.kernel,
      out_type=tuple(jax.ShapeDtypeStruct((N_MOL, DM), jnp.float32)
                     for _ in range(4)),
      mesh=plsc.VectorSubcoreMesh(core_axis_name="c", subcore_axis_name="s",
                                  num_cores=NC, num_subcores=NS),
      scratch_types=[
          pltpu.VMEM((SUP, CHUNK), jnp.int32),
          pltpu.VMEM((SUP, CHUNK), jnp.int32),
          pltpu.VMEM((SUP, CHUNK, DM), jnp.float32),
          pltpu.VMEM((CHUNK, DM), jnp.float32),
          pltpu.VMEM_SHARED((N_MOL, DM), jnp.float32),
          pltpu.SemaphoreType.DMA,
          pltpu.SemaphoreType.DMA,
      ],
      compiler_params=pltpu.CompilerParams(use_tc_tiling_on_sc=False),
  )(_sc_mol_prop_body)


def _sc_mol_prop(molx_pad, src, dst):
  return _build_sc_mol_prop()(molx_pad, src, dst)


def _sc_mol_prop_body(molx_hbm, src_hbm, dst_hbm, h0_hbm, h1_hbm, h2_hbm,
                      h3_hbm, src_v, dst_v, msg_v, zbuf, acc, sem, sem2):
    cid = lax.axis_index("c")
    sid = lax.axis_index("s")
    houts = [h0_hbm, h1_hbm, h2_hbm, h3_hbm]
    tables = [molx_hbm, h0_hbm, h1_hbm, h2_hbm]

    @pl.when(cid == 0)
    def _():
        _zero_vmem_2d(zbuf, CHUNK, DM)
        pltpu.sync_copy(src_hbm.at[pl.ds(sid * SUP, SUP)], src_v)
        pltpu.sync_copy(dst_hbm.at[pl.ds(sid * SUP, SUP)], dst_v)
        for k in range(4):
            pltpu.sync_copy(zbuf, acc.at[pl.ds(2 * sid * 128, 128)])
            pltpu.sync_copy(zbuf, acc.at[pl.ds((2 * sid + 1) * 128, 128)])
            plsc.subcore_barrier()

            gd = [pltpu.async_copy(tables[k].at[dst_v.at[b]], msg_v.at[b],
                                   sem)
                  for b in range(SUP)]
            for g in gd:
                g.wait()
            sd = [pltpu.async_copy(msg_v.at[b], acc.at[src_v.at[b]], sem2,
                                   add=True)
                  for b in range(SUP)]
            for d_ in sd:
                d_.wait()
            plsc.subcore_barrier()

            rpt = N_MOL // NS
            pltpu.sync_copy(acc.at[pl.ds(sid * rpt, rpt)],
                            houts[k].at[pl.ds(sid * rpt, rpt)])


def _tc_conv_body(agg_ref, x_ref, wl_ref, wr_ref, b_ref, out_ref):
    a = agg_ref[...]
    deg = jnp.maximum(a[:, 54:55], 1.0)
    an = a * (1.0 / deg)
    o = (jnp.dot(an, wl_ref[...], preferred_element_type=jnp.float32)
         + jnp.dot(x_ref[...], wr_ref[...], preferred_element_type=jnp.float32)
         + b_ref[...])
    out_ref[...] = jnp.maximum(o, 0.0)


def _tc_conv(agg, x, wlt, wrt, bias):
    n = agg.shape[0]
    bn = 1024
    grid = n // bn
    return pl.pallas_call(
        _tc_conv_body,
        grid=(grid,),
        in_specs=[
            pl.BlockSpec((bn, DP), lambda i: (i, 0)),
            pl.BlockSpec((bn, DP), lambda i: (i, 0)),
            pl.BlockSpec((DP, DP), lambda i: (0, 0)),
            pl.BlockSpec((DP, DP), lambda i: (0, 0)),
            pl.BlockSpec((1, DP), lambda i: (0, 0)),
        ],
        out_specs=pl.BlockSpec((bn, DP), lambda i: (i, 0)),
        out_shape=jax.ShapeDtypeStruct((n, DP), jnp.float32),
    )(agg, x, wlt, wrt, bias)


def _tc_conv2_pool_body(agg_ref, x_ref, b3_ref, wl_ref, wr_ref, bias_ref,
                        out_ref):
    i = pl.program_id(0)
    a = agg_ref[...]
    deg = jnp.maximum(a[:, 54:55], 1.0)
    an = a * (1.0 / deg)
    xt2 = jnp.maximum(
        jnp.dot(an, wl_ref[...], preferred_element_type=jnp.float32)
        + jnp.dot(x_ref[...], wr_ref[...], preferred_element_type=jnp.float32)
        + bias_ref[...], 0.0)
    b = b3_ref[0]
    oh = (jax.lax.broadcasted_iota(jnp.int32, (G, b.shape[1]), 0)
          == b).astype(jnp.float32)
    contrib = jnp.dot(oh, xt2, preferred_element_type=jnp.float32)

    @pl.when(i == 0)
    def _():
        out_ref[...] = jnp.zeros_like(out_ref)

    out_ref[...] += contrib


def _tc_conv2_pool(agg, x, batch3, wlt, wrt, bias):
    n = agg.shape[0]
    bn = 1024
    grid = n // bn
    return pl.pallas_call(
        _tc_conv2_pool_body,
        grid=(grid,),
        in_specs=[
            pl.BlockSpec((bn, DP), lambda i: (i, 0)),
            pl.BlockSpec((bn, DP), lambda i: (i, 0)),
            pl.BlockSpec((1, 1, bn), lambda i: (i, 0, 0)),
            pl.BlockSpec((DP, G), lambda i: (0, 0)),
            pl.BlockSpec((DP, G), lambda i: (0, 0)),
            pl.BlockSpec((1, G), lambda i: (0, 0)),
        ],
        out_specs=pl.BlockSpec((G, G), lambda i: (0, 0)),
        out_shape=jax.ShapeDtypeStruct((G, G), jnp.float32),
    )(agg, x, batch3, wlt, wrt, bias)


def _tc_final_body(h0, h1, h2, h3, molx, molb, pro_pool,
                   mg1, mb1, mg2, mb2, pg1, pb1, pg2, pb2,
                   f1a, f1b, b1, f2, b2, ow, ob, out_ref):
    alpha = 0.05
    q = (1.0 - alpha) / 4.0
    emb = alpha * molx[...] + q * (h0[...] + h1[...] + h2[...] + h3[...])
    colmask = jax.lax.broadcasted_iota(jnp.int32, (N_MOL, DM), 1) == 78
    emb = jnp.where(colmask, 1.0, emb)
    ohT = (molb[...] == jax.lax.broadcasted_iota(jnp.int32, (N_MOL, G), 1)
           ).astype(jnp.float32)
    pm = jax.lax.dot_general(ohT, emb, (((0,), (0,)), ((), ())),
                             preferred_element_type=jnp.float32)
    pm = pm * (1.0 / jnp.maximum(pm[:, 78:79], 1.0))
    xm = jnp.maximum(jnp.dot(pm, mg1[...],
                             preferred_element_type=jnp.float32) + mb1[...],
                     0.0)
    xm = jnp.dot(xm, mg2[...], preferred_element_type=jnp.float32) + mb2[...]

    pp = pro_pool[...]
    pp = pp * (1.0 / jnp.maximum(pp[:, 108:109], 1.0))
    xp = jnp.maximum(jnp.dot(pp, pg1[...],
                             preferred_element_type=jnp.float32) + pb1[...],
                     0.0)
    xp = jnp.dot(xp, pg2[...], preferred_element_type=jnp.float32) + pb2[...]

    h = jnp.maximum(jnp.dot(xm, f1a[...], preferred_element_type=jnp.float32)
                    + jnp.dot(xp, f1b[...],
                              preferred_element_type=jnp.float32)
                    + b1[...], 0.0)
    h = jnp.maximum(jnp.dot(h, f2[...], preferred_element_type=jnp.float32)
                    + b2[...], 0.0)
    out_ref[...] = (jnp.dot(h, ow[...], preferred_element_type=jnp.float32)
                    + ob[...])


def _tc_final(hs, molx, molb2, pro_pool, mg1, mb1, mg2, mb2,
              pg1, pb1, pg2, pb2, f1a, f1b, b1, f2, b2, ow, ob):
    return pl.pallas_call(
        _tc_final_body,
        out_shape=jax.ShapeDtypeStruct((G, 1), jnp.float32),
    )(hs[0], hs[1], hs[2], hs[3], molx, molb2, pro_pool,
      mg1, mb1, mg2, mb2, pg1, pb1, pg2, pb2, f1a, f1b, b1, f2, b2, ow, ob)


def kernel(mol_x, mol_edge_index, mol_batch, target_x, target_edge_index,
           target_batch, mol_fc_g1_W, mol_fc_g1_b, mol_fc_g2_W, mol_fc_g2_b,
           pro_conv1_Wl, pro_conv1_Wr, pro_conv1_b, pro_conv2_Wl,
           pro_conv2_Wr, pro_conv2_b, pro_fc_g1_W, pro_fc_g1_b, pro_fc_g2_W,
           pro_fc_g2_b, fc1_W, fc1_b, fc2_W, fc2_b, out_W, out_b):
    f32 = jnp.float32

    n_pro = target_x.shape[0]
    x_pad = jnp.concatenate(
        [target_x, jnp.ones((n_pro, 1), f32), jnp.zeros((n_pro, DP - 55), f32)],
        axis=1)
    x_pad = jnp.pad(x_pad, ((0, NP_PAD - n_pro), (0, 0)))
    src_pad = jnp.pad(target_edge_index[0],
                      (0, E_PAD - E_PRO)).reshape(E_PAD // CHUNK, CHUNK)
    dst_pad = jnp.pad(target_edge_index[1], (0, E_PAD - E_PRO),
                      constant_values=-1).reshape(E_PAD // CHUNK, CHUNK)
    tb3 = jnp.pad(target_batch, (0, NP_PAD - n_pro),
                  constant_values=-1).reshape(NP_PAD // 1024, 1, 1024)

    molx_pad = jnp.pad(mol_x, ((0, 0), (0, DM - mol_x.shape[1])))
    molb2 = mol_batch.reshape(N_MOL, 1)

    def padw(w, o, i):
        return jnp.pad(w, ((0, o - w.shape[0]), (0, i - w.shape[1]))).T

    wl1 = padw(pro_conv1_Wl, DP, DP)
    wr1 = padw(pro_conv1_Wr, DP, DP)
    b1c = jnp.zeros((DP,), f32).at[:54].set(pro_conv1_b).at[54].set(1.0)
    wl2 = padw(pro_conv2_Wl, G, DP)
    wr2 = padw(pro_conv2_Wr, G, DP)
    b2c = jnp.zeros((G,), f32).at[:108].set(pro_conv2_b).at[108].set(1.0)

    mg1 = jnp.pad(mol_fc_g1_W, ((0, 0), (0, DM - 78))).T
    mg2 = mol_fc_g2_W.T
    pg1 = jnp.pad(pro_fc_g1_W, ((0, 0), (0, G - 108))).T
    pg2 = pro_fc_g2_W.T
    f1a = fc1_W[:, :112].T
    f1b = fc1_W[:, 112:].T
    f2 = fc2_W.T
    ow = out_W.T

    row = lambda v: v.reshape(1, -1)

    hs = _sc_mol_prop(molx_pad,
                      mol_edge_index[0].reshape(E_MOL // CHUNK, CHUNK),
                      mol_edge_index[1].reshape(E_MOL // CHUNK, CHUNK))

    def colstack(v):
        return jnp.concatenate([v[:, :DPH], v[:, DPH:]], axis=0)

    def colunstack(v):
        return jnp.concatenate([v[:NP_PAD], v[NP_PAD:]], axis=1)

    agg1 = colunstack(_sc_conv_agg(colstack(x_pad), src_pad, dst_pad))
    xt1 = _tc_conv(agg1, x_pad, wl1, wr1, row(b1c))
    agg2 = colunstack(_sc_conv_agg(colstack(xt1), src_pad, dst_pad))
    pro_pool = _tc_conv2_pool(agg2, xt1, tb3, wl2, wr2, row(b2c))

    return _tc_final(hs, molx_pad, molb2, pro_pool,
                     mg1, row(mol_fc_g1_b), mg2, row(mol_fc_g2_b),
                     pg1, row(pro_fc_g1_b), pg2, row(pro_fc_g2_b),
                     f1a, f1b, row(fc1_b), f2, row(fc2_b),
                     ow, out_b.reshape(1, 1))
